# Initial kernel scaffold; baseline (speedup 1.0000x reference)
import jax
import jax.numpy as jnp
from jax.experimental import pallas as pl


def kernel(h, coord, edge_index, coord_diff, coord_cross, edge_attr, w0, b0, w1, b1, w2):
    raise NotImplementedError("write your pallas kernel here")


# speedup vs baseline: 4.1498x; 4.1498x over previous
import dataclasses
import functools

import jax
import jax.numpy as jnp
from jax import lax
from jax.experimental import pallas as pl
from jax.experimental.pallas import tpu as pltpu
from jax.experimental.pallas import tpu_sc as plsc

N = 10000
E = 320000
H = 128
NORM = 100.0

NC = 2
NS = 16
NW = NC * NS
EW = E // NW
CH_G = 80
NCH_G = EW // CH_G
CH_S = 2000
NCH_S = EW // CH_S
VL = 16


@functools.lru_cache(maxsize=None)
def _sc_mesh():
    return plsc.VectorSubcoreMesh(core_axis_name="c", subcore_axis_name="s",
                                  num_cores=NC, num_subcores=NS)


@functools.lru_cache(maxsize=None)
def _sc_params():
    cp = pltpu.CompilerParams()
    if "needs_layout_passes" in pltpu.CompilerParams.__dataclass_fields__:
        cp = dataclasses.replace(cp, needs_layout_passes=False)
    return cp


def _proj_body(h_ref, wa_ref, wb_ref, b0_ref, ha_ref, hb_ref):
    hb16 = h_ref[...].astype(jnp.bfloat16)
    ha_ref[...] = (
        jnp.dot(hb16, wa_ref[...], preferred_element_type=jnp.float32)
        + b0_ref[...]
    )
    hb_ref[...] = jnp.dot(hb16, wb_ref[...], preferred_element_type=jnp.float32)


def _project(h, wa_t, wb_t, b0_row):
    blk = 2000
    grid = N // blk
    return pl.pallas_call(
        _proj_body,
        grid=(grid,),
        in_specs=[
            pl.BlockSpec((blk, H), lambda i: (i, 0)),
            pl.BlockSpec((H, H), lambda i: (0, 0)),
            pl.BlockSpec((H, H), lambda i: (0, 0)),
            pl.BlockSpec((1, H), lambda i: (0, 0)),
        ],
        out_specs=[
            pl.BlockSpec((blk, H), lambda i: (i, 0)),
            pl.BlockSpec((blk, H), lambda i: (i, 0)),
        ],
        out_shape=[
            jax.ShapeDtypeStruct((N, H), jnp.float32),
            jax.ShapeDtypeStruct((N, H), jnp.float32),
        ],
    )(h, wa_t, wb_t, b0_row)


def _sc_gather(ha, hb, row, col):
    @functools.partial(
        pl.kernel,
        mesh=_sc_mesh(),
        out_type=[
            jax.ShapeDtypeStruct((E, H), jnp.float32),
            jax.ShapeDtypeStruct((E, H), jnp.float32),
        ],
        scratch_types=[
            pltpu.VMEM((CH_G,), jnp.int32),
            pltpu.VMEM((CH_G,), jnp.int32),
            pltpu.VMEM((CH_G, H), jnp.float32),
            pltpu.VMEM((CH_G, H), jnp.float32),
            pltpu.SemaphoreType.DMA,
            pltpu.SemaphoreType.DMA,
        ],
    )
    def body(ha_hbm, hb_hbm, row_hbm, col_hbm, ga_hbm, gb_hbm,
             idxr_v, idxc_v, bufa_v, bufb_v, sema, semb):
        wid = lax.axis_index("s") * NC + lax.axis_index("c")

        @pl.loop(0, NCH_G)
        def _(ci):
            base = wid * EW + ci * CH_G
            pltpu.sync_copy(row_hbm.at[pl.ds(base, CH_G)], idxr_v)
            pltpu.sync_copy(col_hbm.at[pl.ds(base, CH_G)], idxc_v)
            cpa = pltpu.async_copy(ha_hbm.at[idxr_v], bufa_v, sema)
            cpb = pltpu.async_copy(hb_hbm.at[idxc_v], bufb_v, semb)
            cpa.wait()
            cpb.wait()
            pltpu.sync_copy(bufa_v, ga_hbm.at[pl.ds(base, CH_G)])
            pltpu.sync_copy(bufb_v, gb_hbm.at[pl.ds(base, CH_G)])

    return body(ha, hb, row, col)


def _mlp_body(ga_ref, gb_ref, attr_ref, cdt_ref, w0c_ref, w1t_ref, b1_ref,
              w2_ref, outx_ref, outy_ref, outz_ref):
    pre0 = ga_ref[...] + gb_ref[...] + attr_ref[...] * w0c_ref[...]
    m0 = pre0 * jax.nn.sigmoid(pre0)
    mm = (
        jnp.dot(m0.astype(jnp.bfloat16), w1t_ref[...],
                preferred_element_type=jnp.float32)
        + b1_ref[...]
    )
    m1 = mm * jax.nn.sigmoid(mm)
    phi_t = jax.lax.dot_general(
        w2_ref[...], m1.astype(jnp.bfloat16),
        dimension_numbers=(((1,), (1,)), ((), ())),
        preferred_element_type=jnp.float32)
    t = cdt_ref[...] * phi_t
    base = pl.program_id(0) * t.shape[1]
    outx_ref[pl.ds(base, t.shape[1])] = t[0]
    outy_ref[pl.ds(base, t.shape[1])] = t[1]
    outz_ref[pl.ds(base, t.shape[1])] = t[2]


def _edge_mlp(ga, gb, edge_attr, cdt, w0c_row, w1t, b1_row, w2b):
    blk = 6400
    grid = E // blk
    return pl.pallas_call(
        _mlp_body,
        grid=(grid,),
        in_specs=[
            pl.BlockSpec((blk, H), lambda i: (i, 0)),
            pl.BlockSpec((blk, H), lambda i: (i, 0)),
            pl.BlockSpec((blk, 1), lambda i: (i, 0)),
            pl.BlockSpec((3, blk), lambda i: (0, i)),
            pl.BlockSpec((1, H), lambda i: (0, 0)),
            pl.BlockSpec((H, H), lambda i: (0, 0)),
            pl.BlockSpec((1, H), lambda i: (0, 0)),
            pl.BlockSpec((1, H), lambda i: (0, 0)),
        ],
        out_specs=[
            pl.BlockSpec((E,), lambda i: (0,)),
            pl.BlockSpec((E,), lambda i: (0,)),
            pl.BlockSpec((E,), lambda i: (0,)),
        ],
        out_shape=[
            jax.ShapeDtypeStruct((E,), jnp.float32),
            jax.ShapeDtypeStruct((E,), jnp.float32),
            jax.ShapeDtypeStruct((E,), jnp.float32),
        ],
    )(ga, gb, edge_attr, cdt, w0c_row, w1t, b1_row, w2b)


def _sc_scatter(tx, ty, tz, row):
    @functools.partial(
        pl.kernel,
        mesh=_sc_mesh(),
        compiler_params=_sc_params(),
        out_type=[
            jax.ShapeDtypeStruct((NW, 1, N), jnp.float32),
            jax.ShapeDtypeStruct((NW, 1, N), jnp.float32),
            jax.ShapeDtypeStruct((NW, 1, N), jnp.float32),
        ],
        scratch_types=[
            pltpu.VMEM((CH_S,), jnp.int32),
            pltpu.VMEM((CH_S,), jnp.float32),
            pltpu.VMEM((CH_S,), jnp.float32),
            pltpu.VMEM((CH_S,), jnp.float32),
            pltpu.VMEM((N,), jnp.float32),
            pltpu.VMEM((N,), jnp.float32),
            pltpu.VMEM((N,), jnp.float32),
        ],
    )
    def body(tx_hbm, ty_hbm, tz_hbm, row_hbm, outx_hbm, outy_hbm, outz_hbm,
             idx_v, tx_v, ty_v, tz_v, accx_v, accy_v, accz_v):
        wid = lax.axis_index("s") * NC + lax.axis_index("c")
        zero16 = jnp.zeros((VL,), jnp.float32)

        @pl.loop(0, N, step=VL)
        def _(k):
            accx_v[pl.ds(k, VL)] = zero16
            accy_v[pl.ds(k, VL)] = zero16
            accz_v[pl.ds(k, VL)] = zero16

        @pl.loop(0, NCH_S)
        def _(ci):
            base = wid * EW + ci * CH_S
            pltpu.sync_copy(row_hbm.at[pl.ds(base, CH_S)], idx_v)
            pltpu.sync_copy(tx_hbm.at[pl.ds(base, CH_S)], tx_v)
            pltpu.sync_copy(ty_hbm.at[pl.ds(base, CH_S)], ty_v)
            pltpu.sync_copy(tz_hbm.at[pl.ds(base, CH_S)], tz_v)

            @pl.loop(0, CH_S, step=VL)
            def _(k):
                i16 = idx_v[pl.ds(k, VL)]
                plsc.addupdate_scatter(accx_v, [i16], tx_v[pl.ds(k, VL)])
                plsc.addupdate_scatter(accy_v, [i16], ty_v[pl.ds(k, VL)])
                plsc.addupdate_scatter(accz_v, [i16], tz_v[pl.ds(k, VL)])

        pltpu.sync_copy(accx_v, outx_hbm.at[wid, 0])
        pltpu.sync_copy(accy_v, outy_hbm.at[wid, 0])
        pltpu.sync_copy(accz_v, outz_hbm.at[wid, 0])

    return body(tx, ty, tz, row)


def _combine_body(coordt_ref, px_ref, py_ref, pz_ref, out_ref):
    agg = jnp.concatenate(
        [jnp.sum(px_ref[...], axis=0, keepdims=True),
         jnp.sum(py_ref[...], axis=0, keepdims=True),
         jnp.sum(pz_ref[...], axis=0, keepdims=True)], axis=0)
    out_ref[...] = coordt_ref[...] + agg * (1.0 / NORM)


def _combine(coordt, px, py, pz):
    return pl.pallas_call(
        _combine_body,
        in_specs=[
            pl.BlockSpec((3, N), lambda: (0, 0)),
            pl.BlockSpec((NW, N), lambda: (0, 0)),
            pl.BlockSpec((NW, N), lambda: (0, 0)),
            pl.BlockSpec((NW, N), lambda: (0, 0)),
        ],
        out_specs=pl.BlockSpec((3, N), lambda: (0, 0)),
        out_shape=jax.ShapeDtypeStruct((3, N), jnp.float32),
    )(coordt, px, py, pz)


def kernel(h, coord, edge_index, coord_diff, coord_cross, edge_attr,
           w0, b0, w1, b1, w2):
    row = edge_index[0]
    col = edge_index[1]
    wa_t = jnp.transpose(w0[:, :H]).astype(jnp.bfloat16)
    wb_t = jnp.transpose(w0[:, H:2 * H]).astype(jnp.bfloat16)
    w0c_row = jnp.transpose(w0[:, 2 * H:])
    w1t = jnp.transpose(w1).astype(jnp.bfloat16)
    w2b = w2.astype(jnp.bfloat16)
    b0_row = b0.reshape(1, H)
    b1_row = b1.reshape(1, H)
    cdt = jnp.transpose(coord_diff)
    coordt = jnp.transpose(coord)

    ha, hb = _project(h, wa_t, wb_t, b0_row)
    ga, gb = _sc_gather(ha, hb, row, col)
    tx, ty, tz = _edge_mlp(ga, gb, edge_attr, cdt, w0c_row, w1t, b1_row, w2b)
    px, py, pz = _sc_scatter(tx, ty, tz, row)
    out_t = _combine(coordt, px.reshape(NW, N), py.reshape(NW, N),
                     pz.reshape(NW, N))
    return jnp.transpose(out_t)


# speedup vs baseline: 5.2335x; 1.2612x over previous
import dataclasses
import functools

import jax
import jax.numpy as jnp
from jax import lax
from jax.experimental import pallas as pl
from jax.experimental.pallas import tpu as pltpu
from jax.experimental.pallas import tpu_sc as plsc

N = 10000
E = 320000
H = 128
NORM = 100.0

NC = 2
NS = 16
NW = NC * NS
EW = E // NW
CH_G = 80
NCH_G = EW // CH_G
CH_S = 2000
NCH_S = EW // CH_S
VL = 16


@functools.lru_cache(maxsize=None)
def _sc_mesh():
    return plsc.VectorSubcoreMesh(core_axis_name="c", subcore_axis_name="s",
                                  num_cores=NC, num_subcores=NS)


@functools.lru_cache(maxsize=None)
def _sc_params():
    cp = pltpu.CompilerParams()
    if "needs_layout_passes" in pltpu.CompilerParams.__dataclass_fields__:
        cp = dataclasses.replace(cp, needs_layout_passes=False)
    return cp


def _proj_body(h_ref, wa_ref, wb_ref, b0_ref, ha_ref, hb_ref):
    hb16 = h_ref[...].astype(jnp.bfloat16)
    ha_ref[...] = (
        jnp.dot(hb16, wa_ref[...], preferred_element_type=jnp.float32)
        + b0_ref[...]
    )
    hb_ref[...] = jnp.dot(hb16, wb_ref[...], preferred_element_type=jnp.float32)


def _project(h, wa_t, wb_t, b0_row):
    blk = 2000
    grid = N // blk
    return pl.pallas_call(
        _proj_body,
        grid=(grid,),
        in_specs=[
            pl.BlockSpec((blk, H), lambda i: (i, 0)),
            pl.BlockSpec((H, H), lambda i: (0, 0)),
            pl.BlockSpec((H, H), lambda i: (0, 0)),
            pl.BlockSpec((1, H), lambda i: (0, 0)),
        ],
        out_specs=[
            pl.BlockSpec((blk, H), lambda i: (i, 0)),
            pl.BlockSpec((blk, H), lambda i: (i, 0)),
        ],
        out_shape=[
            jax.ShapeDtypeStruct((N, H), jnp.float32),
            jax.ShapeDtypeStruct((N, H), jnp.float32),
        ],
    )(h, wa_t, wb_t, b0_row)


KG = 4
NCH_MAIN = (NCH_G // KG) * KG


def _sc_gather(ha, hb, row, col):
    scratch = []
    for _ in range(KG):
        scratch += [pltpu.VMEM((CH_G,), jnp.int32),
                    pltpu.VMEM((CH_G,), jnp.int32),
                    pltpu.VMEM((CH_G, H), jnp.float32),
                    pltpu.VMEM((CH_G, H), jnp.float32)]
    scratch += [pltpu.SemaphoreType.DMA] * (3 * KG)

    @functools.partial(
        pl.kernel,
        mesh=_sc_mesh(),
        out_type=[
            jax.ShapeDtypeStruct((E, H), jnp.float32),
            jax.ShapeDtypeStruct((E, H), jnp.float32),
        ],
        scratch_types=scratch,
    )
    def body(ha_hbm, hb_hbm, row_hbm, col_hbm, ga_hbm, gb_hbm, *s):
        idxr = [s[4 * b] for b in range(KG)]
        idxc = [s[4 * b + 1] for b in range(KG)]
        bufa = [s[4 * b + 2] for b in range(KG)]
        bufb = [s[4 * b + 3] for b in range(KG)]
        semi = [s[4 * KG + b] for b in range(KG)]
        semg = [s[5 * KG + b] for b in range(KG)]
        semw = [s[6 * KG + b] for b in range(KG)]
        wid = lax.axis_index("s") * NC + lax.axis_index("c")

        def chunk_base(ci, b):
            return wid * EW + (ci + b) * CH_G

        @pl.loop(0, NCH_MAIN, step=KG)
        def _(ci):
            hi = []
            for b in range(KG):
                base = chunk_base(ci, b)
                hi.append((
                    pltpu.async_copy(row_hbm.at[pl.ds(base, CH_G)],
                                     idxr[b], semi[b]),
                    pltpu.async_copy(col_hbm.at[pl.ds(base, CH_G)],
                                     idxc[b], semi[b]),
                ))
            hg = []
            for b in range(KG):
                hi[b][0].wait()
                hi[b][1].wait()
                hg.append((
                    pltpu.async_copy(ha_hbm.at[idxr[b]], bufa[b], semg[b]),
                    pltpu.async_copy(hb_hbm.at[idxc[b]], bufb[b], semg[b]),
                ))
            hw = []
            for b in range(KG):
                base = chunk_base(ci, b)
                hg[b][0].wait()
                hg[b][1].wait()
                hw.append((
                    pltpu.async_copy(bufa[b], ga_hbm.at[pl.ds(base, CH_G)],
                                     semw[b]),
                    pltpu.async_copy(bufb[b], gb_hbm.at[pl.ds(base, CH_G)],
                                     semw[b]),
                ))
            for b in range(KG):
                hw[b][0].wait()
                hw[b][1].wait()

        @pl.loop(NCH_MAIN, NCH_G)
        def _(ci):
            base = wid * EW + ci * CH_G
            pltpu.sync_copy(row_hbm.at[pl.ds(base, CH_G)], idxr[0])
            pltpu.sync_copy(col_hbm.at[pl.ds(base, CH_G)], idxc[0])
            cpa = pltpu.async_copy(ha_hbm.at[idxr[0]], bufa[0], semg[0])
            cpb = pltpu.async_copy(hb_hbm.at[idxc[0]], bufb[0], semg[0])
            cpa.wait()
            cpb.wait()
            pltpu.sync_copy(bufa[0], ga_hbm.at[pl.ds(base, CH_G)])
            pltpu.sync_copy(bufb[0], gb_hbm.at[pl.ds(base, CH_G)])

    return body(ha, hb, row, col)


def _mlp_body(ga_ref, gb_ref, attr_ref, cdt_ref, w0c_ref, w1t_ref, b1_ref,
              w2_ref, outx_ref, outy_ref, outz_ref):
    pre0 = ga_ref[...] + gb_ref[...] + attr_ref[...] * w0c_ref[...]
    m0 = pre0 * jax.nn.sigmoid(pre0)
    mm = (
        jnp.dot(m0.astype(jnp.bfloat16), w1t_ref[...],
                preferred_element_type=jnp.float32)
        + b1_ref[...]
    )
    m1 = mm * jax.nn.sigmoid(mm)
    phi_t = jax.lax.dot_general(
        w2_ref[...], m1.astype(jnp.bfloat16),
        dimension_numbers=(((1,), (1,)), ((), ())),
        preferred_element_type=jnp.float32)
    t = cdt_ref[...] * phi_t
    base = pl.program_id(0) * t.shape[1]
    outx_ref[pl.ds(base, t.shape[1])] = t[0]
    outy_ref[pl.ds(base, t.shape[1])] = t[1]
    outz_ref[pl.ds(base, t.shape[1])] = t[2]


def _edge_mlp(ga, gb, edge_attr, cdt, w0c_row, w1t, b1_row, w2b):
    blk = 6400
    grid = E // blk
    return pl.pallas_call(
        _mlp_body,
        grid=(grid,),
        in_specs=[
            pl.BlockSpec((blk, H), lambda i: (i, 0)),
            pl.BlockSpec((blk, H), lambda i: (i, 0)),
            pl.BlockSpec((blk, 1), lambda i: (i, 0)),
            pl.BlockSpec((3, blk), lambda i: (0, i)),
            pl.BlockSpec((1, H), lambda i: (0, 0)),
            pl.BlockSpec((H, H), lambda i: (0, 0)),
            pl.BlockSpec((1, H), lambda i: (0, 0)),
            pl.BlockSpec((1, H), lambda i: (0, 0)),
        ],
        out_specs=[
            pl.BlockSpec((E,), lambda i: (0,)),
            pl.BlockSpec((E,), lambda i: (0,)),
            pl.BlockSpec((E,), lambda i: (0,)),
        ],
        out_shape=[
            jax.ShapeDtypeStruct((E,), jnp.float32),
            jax.ShapeDtypeStruct((E,), jnp.float32),
            jax.ShapeDtypeStruct((E,), jnp.float32),
        ],
    )(ga, gb, edge_attr, cdt, w0c_row, w1t, b1_row, w2b)


def _sc_scatter(tx, ty, tz, row):
    @functools.partial(
        pl.kernel,
        mesh=_sc_mesh(),
        compiler_params=_sc_params(),
        out_type=[
            jax.ShapeDtypeStruct((NW, 1, N), jnp.float32),
            jax.ShapeDtypeStruct((NW, 1, N), jnp.float32),
            jax.ShapeDtypeStruct((NW, 1, N), jnp.float32),
        ],
        scratch_types=[
            pltpu.VMEM((CH_S,), jnp.int32),
            pltpu.VMEM((CH_S,), jnp.float32),
            pltpu.VMEM((CH_S,), jnp.float32),
            pltpu.VMEM((CH_S,), jnp.float32),
            pltpu.VMEM((N,), jnp.float32),
            pltpu.VMEM((N,), jnp.float32),
            pltpu.VMEM((N,), jnp.float32),
        ],
    )
    def body(tx_hbm, ty_hbm, tz_hbm, row_hbm, outx_hbm, outy_hbm, outz_hbm,
             idx_v, tx_v, ty_v, tz_v, accx_v, accy_v, accz_v):
        wid = lax.axis_index("s") * NC + lax.axis_index("c")
        zero16 = jnp.zeros((VL,), jnp.float32)

        @pl.loop(0, N, step=VL)
        def _(k):
            accx_v[pl.ds(k, VL)] = zero16
            accy_v[pl.ds(k, VL)] = zero16
            accz_v[pl.ds(k, VL)] = zero16

        @pl.loop(0, NCH_S)
        def _(ci):
            base = wid * EW + ci * CH_S
            pltpu.sync_copy(row_hbm.at[pl.ds(base, CH_S)], idx_v)
            pltpu.sync_copy(tx_hbm.at[pl.ds(base, CH_S)], tx_v)
            pltpu.sync_copy(ty_hbm.at[pl.ds(base, CH_S)], ty_v)
            pltpu.sync_copy(tz_hbm.at[pl.ds(base, CH_S)], tz_v)

            @pl.loop(0, CH_S, step=VL)
            def _(k):
                i16 = idx_v[pl.ds(k, VL)]
                plsc.addupdate_scatter(accx_v, [i16], tx_v[pl.ds(k, VL)])
                plsc.addupdate_scatter(accy_v, [i16], ty_v[pl.ds(k, VL)])
                plsc.addupdate_scatter(accz_v, [i16], tz_v[pl.ds(k, VL)])

        pltpu.sync_copy(accx_v, outx_hbm.at[wid, 0])
        pltpu.sync_copy(accy_v, outy_hbm.at[wid, 0])
        pltpu.sync_copy(accz_v, outz_hbm.at[wid, 0])

    return body(tx, ty, tz, row)


def _combine_body(coordt_ref, px_ref, py_ref, pz_ref, out_ref):
    agg = jnp.concatenate(
        [jnp.sum(px_ref[...], axis=0, keepdims=True),
         jnp.sum(py_ref[...], axis=0, keepdims=True),
         jnp.sum(pz_ref[...], axis=0, keepdims=True)], axis=0)
    out_ref[...] = coordt_ref[...] + agg * (1.0 / NORM)


def _combine(coordt, px, py, pz):
    return pl.pallas_call(
        _combine_body,
        in_specs=[
            pl.BlockSpec((3, N), lambda: (0, 0)),
            pl.BlockSpec((NW, N), lambda: (0, 0)),
            pl.BlockSpec((NW, N), lambda: (0, 0)),
            pl.BlockSpec((NW, N), lambda: (0, 0)),
        ],
        out_specs=pl.BlockSpec((3, N), lambda: (0, 0)),
        out_shape=jax.ShapeDtypeStruct((3, N), jnp.float32),
    )(coordt, px, py, pz)


def kernel(h, coord, edge_index, coord_diff, coord_cross, edge_attr,
           w0, b0, w1, b1, w2):
    row = edge_index[0]
    col = edge_index[1]
    wa_t = jnp.transpose(w0[:, :H]).astype(jnp.bfloat16)
    wb_t = jnp.transpose(w0[:, H:2 * H]).astype(jnp.bfloat16)
    w0c_row = jnp.transpose(w0[:, 2 * H:])
    w1t = jnp.transpose(w1).astype(jnp.bfloat16)
    w2b = w2.astype(jnp.bfloat16)
    b0_row = b0.reshape(1, H)
    b1_row = b1.reshape(1, H)
    cdt = jnp.transpose(coord_diff)
    coordt = jnp.transpose(coord)

    ha, hb = _project(h, wa_t, wb_t, b0_row)
    ga, gb = _sc_gather(ha, hb, row, col)
    tx, ty, tz = _edge_mlp(ga, gb, edge_attr, cdt, w0c_row, w1t, b1_row, w2b)
    px, py, pz = _sc_scatter(tx, ty, tz, row)
    out_t = _combine(coordt, px.reshape(NW, N), py.reshape(NW, N),
                     pz.reshape(NW, N))
    return jnp.transpose(out_t)


# speedup vs baseline: 5.2698x; 1.0069x over previous
import dataclasses
import functools

import jax
import jax.numpy as jnp
from jax import lax
from jax.experimental import pallas as pl
from jax.experimental.pallas import tpu as pltpu
from jax.experimental.pallas import tpu_sc as plsc

N = 10000
E = 320000
H = 128
NORM = 100.0

NC = 2
NS = 16
NW = NC * NS
EW = E // NW
NSLICE = 2
ES = E // NSLICE
EWS = ES // NW
CH_G = 40
NCH_G = EWS // CH_G
CH_S = 2000
NCH_S = EW // CH_S
VL = 16


@functools.lru_cache(maxsize=None)
def _sc_mesh():
    return plsc.VectorSubcoreMesh(core_axis_name="c", subcore_axis_name="s",
                                  num_cores=NC, num_subcores=NS)


@functools.lru_cache(maxsize=None)
def _sc_params():
    cp = pltpu.CompilerParams()
    if "needs_layout_passes" in pltpu.CompilerParams.__dataclass_fields__:
        cp = dataclasses.replace(cp, needs_layout_passes=False)
    return cp


def _proj_body(h_ref, wa_ref, wb_ref, b0_ref, ha_ref, hb_ref):
    hb16 = h_ref[...].astype(jnp.bfloat16)
    ha_ref[...] = (
        jnp.dot(hb16, wa_ref[...], preferred_element_type=jnp.float32)
        + b0_ref[...]
    )
    hb_ref[...] = jnp.dot(hb16, wb_ref[...], preferred_element_type=jnp.float32)


def _project(h, wa_t, wb_t, b0_row):
    blk = 2000
    grid = N // blk
    return pl.pallas_call(
        _proj_body,
        grid=(grid,),
        in_specs=[
            pl.BlockSpec((blk, H), lambda i: (i, 0)),
            pl.BlockSpec((H, H), lambda i: (0, 0)),
            pl.BlockSpec((H, H), lambda i: (0, 0)),
            pl.BlockSpec((1, H), lambda i: (0, 0)),
        ],
        out_specs=[
            pl.BlockSpec((blk, H), lambda i: (i, 0)),
            pl.BlockSpec((blk, H), lambda i: (i, 0)),
        ],
        out_shape=[
            jax.ShapeDtypeStruct((N, H), jnp.float32),
            jax.ShapeDtypeStruct((N, H), jnp.float32),
        ],
    )(h, wa_t, wb_t, b0_row)


KG = 4
NCH_MAIN = (NCH_G // KG) * KG


def _sc_gather(ha, hb, row, col, e0):
    scratch = []
    for _ in range(KG):
        scratch += [pltpu.VMEM((CH_G,), jnp.int32),
                    pltpu.VMEM((CH_G,), jnp.int32),
                    pltpu.VMEM((CH_G, H), jnp.float32),
                    pltpu.VMEM((CH_G, H), jnp.float32)]
    scratch += [pltpu.SemaphoreType.DMA] * (3 * KG)

    @functools.partial(
        pl.kernel,
        mesh=_sc_mesh(),
        out_type=[
            jax.ShapeDtypeStruct((ES, H), jnp.float32),
            jax.ShapeDtypeStruct((ES, H), jnp.float32),
        ],
        scratch_types=scratch,
    )
    def body(ha_hbm, hb_hbm, row_hbm, col_hbm, ga_hbm, gb_hbm, *s):
        idxr = [s[4 * b] for b in range(KG)]
        idxc = [s[4 * b + 1] for b in range(KG)]
        bufa = [s[4 * b + 2] for b in range(KG)]
        bufb = [s[4 * b + 3] for b in range(KG)]
        semi = [s[4 * KG + b] for b in range(KG)]
        semg = [s[5 * KG + b] for b in range(KG)]
        semw = [s[6 * KG + b] for b in range(KG)]
        wid = lax.axis_index("s") * NC + lax.axis_index("c")

        def chunk_base(ci, b):
            return wid * EWS + (ci + b) * CH_G

        @pl.loop(0, NCH_MAIN, step=KG)
        def _(ci):
            hi = []
            for b in range(KG):
                base = chunk_base(ci, b)
                hi.append((
                    pltpu.async_copy(row_hbm.at[pl.ds(e0 + base, CH_G)],
                                     idxr[b], semi[b]),
                    pltpu.async_copy(col_hbm.at[pl.ds(e0 + base, CH_G)],
                                     idxc[b], semi[b]),
                ))
            hg = []
            for b in range(KG):
                hi[b][0].wait()
                hi[b][1].wait()
                hg.append((
                    pltpu.async_copy(ha_hbm.at[idxr[b]], bufa[b], semg[b]),
                    pltpu.async_copy(hb_hbm.at[idxc[b]], bufb[b], semg[b]),
                ))
            hw = []
            for b in range(KG):
                base = chunk_base(ci, b)
                hg[b][0].wait()
                hg[b][1].wait()
                hw.append((
                    pltpu.async_copy(bufa[b], ga_hbm.at[pl.ds(base, CH_G)],
                                     semw[b]),
                    pltpu.async_copy(bufb[b], gb_hbm.at[pl.ds(base, CH_G)],
                                     semw[b]),
                ))
            for b in range(KG):
                hw[b][0].wait()
                hw[b][1].wait()

        @pl.loop(NCH_MAIN, NCH_G)
        def _(ci):
            base = wid * EWS + ci * CH_G
            pltpu.sync_copy(row_hbm.at[pl.ds(e0 + base, CH_G)], idxr[0])
            pltpu.sync_copy(col_hbm.at[pl.ds(e0 + base, CH_G)], idxc[0])
            cpa = pltpu.async_copy(ha_hbm.at[idxr[0]], bufa[0], semg[0])
            cpb = pltpu.async_copy(hb_hbm.at[idxc[0]], bufb[0], semg[0])
            cpa.wait()
            cpb.wait()
            pltpu.sync_copy(bufa[0], ga_hbm.at[pl.ds(base, CH_G)])
            pltpu.sync_copy(bufb[0], gb_hbm.at[pl.ds(base, CH_G)])

    return body(ha, hb, row, col)


def _mlp_body(ga_ref, gb_ref, attr_ref, cdt_ref, w0c_ref, w1t_ref, b1_ref,
              w2_ref, outx_ref, outy_ref, outz_ref):
    pre0 = (ga_ref[...].astype(jnp.float32) + gb_ref[...].astype(jnp.float32)
            + attr_ref[...] * w0c_ref[...])
    m0 = pre0 * jax.nn.sigmoid(pre0)
    mm = (
        jnp.dot(m0.astype(jnp.bfloat16), w1t_ref[...],
                preferred_element_type=jnp.float32)
        + b1_ref[...]
    )
    m1 = mm * jax.nn.sigmoid(mm)
    phi_t = jax.lax.dot_general(
        w2_ref[...], m1.astype(jnp.bfloat16),
        dimension_numbers=(((1,), (1,)), ((), ())),
        preferred_element_type=jnp.float32)
    t = cdt_ref[...] * phi_t
    base = pl.program_id(0) * t.shape[1]
    outx_ref[pl.ds(base, t.shape[1])] = t[0]
    outy_ref[pl.ds(base, t.shape[1])] = t[1]
    outz_ref[pl.ds(base, t.shape[1])] = t[2]


def _edge_mlp(ga, gb, edge_attr, cdt, w0c_row, w1t, b1_row, w2b, sl):
    blk = 6400
    grid = ES // blk
    off = sl * grid
    return pl.pallas_call(
        _mlp_body,
        grid=(grid,),
        in_specs=[
            pl.BlockSpec((blk, H), lambda i: (i, 0)),
            pl.BlockSpec((blk, H), lambda i: (i, 0)),
            pl.BlockSpec((blk, 1), lambda i: (i + off, 0)),
            pl.BlockSpec((3, blk), lambda i: (0, i + off)),
            pl.BlockSpec((1, H), lambda i: (0, 0)),
            pl.BlockSpec((H, H), lambda i: (0, 0)),
            pl.BlockSpec((1, H), lambda i: (0, 0)),
            pl.BlockSpec((1, H), lambda i: (0, 0)),
        ],
        out_specs=[
            pl.BlockSpec((ES,), lambda i: (0,)),
            pl.BlockSpec((ES,), lambda i: (0,)),
            pl.BlockSpec((ES,), lambda i: (0,)),
        ],
        out_shape=[
            jax.ShapeDtypeStruct((ES,), jnp.float32),
            jax.ShapeDtypeStruct((ES,), jnp.float32),
            jax.ShapeDtypeStruct((ES,), jnp.float32),
        ],
    )(ga, gb, edge_attr, cdt, w0c_row, w1t, b1_row, w2b)


def _sc_scatter(tx, ty, tz, row):
    @functools.partial(
        pl.kernel,
        mesh=_sc_mesh(),
        compiler_params=_sc_params(),
        out_type=[
            jax.ShapeDtypeStruct((NW, 1, N), jnp.float32),
            jax.ShapeDtypeStruct((NW, 1, N), jnp.float32),
            jax.ShapeDtypeStruct((NW, 1, N), jnp.float32),
        ],
        scratch_types=[
            pltpu.VMEM((CH_S,), jnp.int32),
            pltpu.VMEM((CH_S,), jnp.float32),
            pltpu.VMEM((CH_S,), jnp.float32),
            pltpu.VMEM((CH_S,), jnp.float32),
            pltpu.VMEM((N,), jnp.float32),
            pltpu.VMEM((N,), jnp.float32),
            pltpu.VMEM((N,), jnp.float32),
        ],
    )
    def body(tx_hbm, ty_hbm, tz_hbm, row_hbm, outx_hbm, outy_hbm, outz_hbm,
             idx_v, tx_v, ty_v, tz_v, accx_v, accy_v, accz_v):
        wid = lax.axis_index("s") * NC + lax.axis_index("c")
        zero16 = jnp.zeros((VL,), jnp.float32)

        @pl.loop(0, N, step=VL)
        def _(k):
            accx_v[pl.ds(k, VL)] = zero16
            accy_v[pl.ds(k, VL)] = zero16
            accz_v[pl.ds(k, VL)] = zero16

        @pl.loop(0, NCH_S)
        def _(ci):
            base = wid * EW + ci * CH_S
            pltpu.sync_copy(row_hbm.at[pl.ds(base, CH_S)], idx_v)
            pltpu.sync_copy(tx_hbm.at[pl.ds(base, CH_S)], tx_v)
            pltpu.sync_copy(ty_hbm.at[pl.ds(base, CH_S)], ty_v)
            pltpu.sync_copy(tz_hbm.at[pl.ds(base, CH_S)], tz_v)

            @pl.loop(0, CH_S, step=VL)
            def _(k):
                i16 = idx_v[pl.ds(k, VL)]
                plsc.addupdate_scatter(accx_v, [i16], tx_v[pl.ds(k, VL)])
                plsc.addupdate_scatter(accy_v, [i16], ty_v[pl.ds(k, VL)])
                plsc.addupdate_scatter(accz_v, [i16], tz_v[pl.ds(k, VL)])

        pltpu.sync_copy(accx_v, outx_hbm.at[wid, 0])
        pltpu.sync_copy(accy_v, outy_hbm.at[wid, 0])
        pltpu.sync_copy(accz_v, outz_hbm.at[wid, 0])

    return body(tx, ty, tz, row)


def _combine_body(coordt_ref, px_ref, py_ref, pz_ref, out_ref):
    agg = jnp.concatenate(
        [jnp.sum(px_ref[...], axis=0, keepdims=True),
         jnp.sum(py_ref[...], axis=0, keepdims=True),
         jnp.sum(pz_ref[...], axis=0, keepdims=True)], axis=0)
    out_ref[...] = coordt_ref[...] + agg * (1.0 / NORM)


def _combine(coordt, px, py, pz):
    return pl.pallas_call(
        _combine_body,
        in_specs=[
            pl.BlockSpec((3, N), lambda: (0, 0)),
            pl.BlockSpec((NW, N), lambda: (0, 0)),
            pl.BlockSpec((NW, N), lambda: (0, 0)),
            pl.BlockSpec((NW, N), lambda: (0, 0)),
        ],
        out_specs=pl.BlockSpec((3, N), lambda: (0, 0)),
        out_shape=jax.ShapeDtypeStruct((3, N), jnp.float32),
    )(coordt, px, py, pz)


def kernel(h, coord, edge_index, coord_diff, coord_cross, edge_attr,
           w0, b0, w1, b1, w2):
    row = edge_index[0]
    col = edge_index[1]
    wa_t = jnp.transpose(w0[:, :H]).astype(jnp.bfloat16)
    wb_t = jnp.transpose(w0[:, H:2 * H]).astype(jnp.bfloat16)
    w0c_row = jnp.transpose(w0[:, 2 * H:])
    w1t = jnp.transpose(w1).astype(jnp.bfloat16)
    w2b = w2.astype(jnp.bfloat16)
    b0_row = b0.reshape(1, H)
    b1_row = b1.reshape(1, H)
    cdt = jnp.transpose(coord_diff)
    coordt = jnp.transpose(coord)

    ha, hb = _project(h, wa_t, wb_t, b0_row)

    txs, tys, tzs = [], [], []
    for sl in range(NSLICE):
        ga, gb = _sc_gather(ha, hb, row, col, sl * ES)
        tx_s, ty_s, tz_s = _edge_mlp(ga, gb, edge_attr, cdt, w0c_row, w1t,
                                     b1_row, w2b, sl)
        txs.append(tx_s)
        tys.append(ty_s)
        tzs.append(tz_s)
    tx = jnp.concatenate(txs)
    ty = jnp.concatenate(tys)
    tz = jnp.concatenate(tzs)
    px, py, pz = _sc_scatter(tx, ty, tz, row)
    out_t = _combine(coordt, px.reshape(NW, N), py.reshape(NW, N),
                     pz.reshape(NW, N))
    return jnp.transpose(out_t)
